# baseline (speedup 1.0000x reference)
import functools

import jax
import jax.numpy as jnp
from jax import lax
from jax.experimental import pallas as pl
from jax.experimental.pallas import tpu as pltpu
from jax.experimental.pallas import tpu_sc as plsc

_C = 21
_TBL = 448


def _argmax_idx_body(x_ref, y_ref, o_ref):
    m = x_ref[0, 0]
    for c in range(1, _C):
        m = jnp.maximum(m, x_ref[0, c])
    a = jnp.where(x_ref[0, 0] == m, 0.0, float(_C))
    for c in range(1, _C):
        a = jnp.minimum(a, jnp.where(x_ref[0, c] == m, float(c), float(_C)))
    o_ref[0] = y_ref[0] * _C + a.astype(jnp.int32)


def _cost_index(inp, y, b0, nb):
    _, c, h, w = inp.shape
    hb = 256
    return pl.pallas_call(
        _argmax_idx_body,
        grid=(nb, h // hb),
        in_specs=[
            pl.BlockSpec((1, c, hb, w), lambda i, j: (b0 + i, 0, j, 0)),
            pl.BlockSpec((1, hb, w), lambda i, j: (b0 + i, j, 0)),
        ],
        out_specs=pl.BlockSpec((1, hb, w), lambda i, j: (i, j, 0)),
        out_shape=jax.ShapeDtypeStruct((nb, h, w), jnp.int32),
    )(inp, y)


def _make_sc_reduce(n):
    info = plsc.get_sparse_core_info()
    nc, ns, lanes = info.num_cores, info.num_subcores, info.num_lanes
    nw = nc * ns
    per_w = n // nw
    mesh = plsc.VectorSubcoreMesh(core_axis_name="c", subcore_axis_name="s")

    @functools.partial(
        pl.kernel,
        mesh=mesh,
        compiler_params=pltpu.CompilerParams(needs_layout_passes=False),
        out_type=jax.ShapeDtypeStruct((nw * lanes,), jnp.float32),
        scratch_types=[
            pltpu.VMEM((per_w,), jnp.int32),
            pltpu.VMEM((_TBL,), jnp.float32),
            pltpu.VMEM((lanes,), jnp.float32),
        ],
    )
    def sc_reduce(idx_hbm, tbl_hbm, out_hbm, idx_v, tbl_v, acc_v):
        wid = lax.axis_index("s") * nc + lax.axis_index("c")
        pltpu.sync_copy(tbl_hbm, tbl_v)
        pltpu.sync_copy(idx_hbm.at[pl.ds(wid * per_w, per_w)], idx_v)

        unroll = 4

        def body(j, accs):
            base = j * (unroll * lanes)
            return tuple(
                accs[u] + plsc.load_gather(
                    tbl_v, [idx_v[pl.ds(base + u * lanes, lanes)]])
                for u in range(unroll)
            )

        z = jnp.zeros((lanes,), jnp.float32)
        accs = lax.fori_loop(0, per_w // (unroll * lanes), body,
                             (z,) * unroll)
        acc_v[...] = (accs[0] + accs[1]) + (accs[2] + accs[3])
        pltpu.sync_copy(acc_v, out_hbm.at[pl.ds(wid * lanes, lanes)])

    return sc_reduce


def kernel(input, y_true, D):
    b, c, h, w = input.shape
    n = b * h * w
    tbl = jnp.zeros((_TBL,), jnp.float32).at[: c * c].set(D.reshape(-1))
    nchunks = 2
    bc = b // nchunks
    sc_reduce = _make_sc_reduce(bc * h * w)
    partials = []
    for k in range(nchunks):
        idx = _cost_index(input, y_true, k * bc, bc).reshape(bc * h * w)
        partials.append(sc_reduce(idx, tbl))
    return jnp.sum(jnp.stack(partials)) / n


# speedup vs baseline: 1.1487x; 1.1487x over previous
import functools

import jax
import jax.numpy as jnp
from jax import lax
from jax.experimental import pallas as pl
from jax.experimental.pallas import tpu as pltpu
from jax.experimental.pallas import tpu_sc as plsc

_C = 21
_TBL = 448


def _argmax_idx_body(x_ref, y_ref, o_ref):
    m = x_ref[0, 0]
    for c in range(1, _C):
        m = jnp.maximum(m, x_ref[0, c])
    a = jnp.where(x_ref[0, 0] == m, 0.0, float(_C))
    for c in range(1, _C):
        a = jnp.minimum(a, jnp.where(x_ref[0, c] == m, float(c), float(_C)))
    o_ref[0] = (y_ref[0] * _C + a.astype(jnp.int32)).astype(jnp.int16)


def _cost_index(inp, y):
    b, c, h, w = inp.shape
    hb = 256
    return pl.pallas_call(
        _argmax_idx_body,
        grid=(b, h // hb),
        in_specs=[
            pl.BlockSpec((1, c, hb, w), lambda i, j: (i, 0, j, 0)),
            pl.BlockSpec((1, hb, w), lambda i, j: (i, j, 0)),
        ],
        out_specs=pl.BlockSpec((1, hb, w), lambda i, j: (i, j, 0)),
        out_shape=jax.ShapeDtypeStruct((b, h, w), jnp.int16),
    )(inp, y)


def _make_sc_reduce(n):
    info = plsc.get_sparse_core_info()
    nc, ns, lanes = info.num_cores, info.num_subcores, info.num_lanes
    nw = nc * ns
    per_w = n // nw
    mesh = plsc.VectorSubcoreMesh(core_axis_name="c", subcore_axis_name="s")

    @functools.partial(
        pl.kernel,
        mesh=mesh,
        compiler_params=pltpu.CompilerParams(needs_layout_passes=False),
        out_type=jax.ShapeDtypeStruct((nw * lanes,), jnp.float32),
        scratch_types=[
            pltpu.VMEM((per_w,), jnp.int16),
            pltpu.VMEM((_TBL,), jnp.float32),
            pltpu.VMEM((lanes,), jnp.float32),
        ],
    )
    def sc_reduce(idx_hbm, tbl_hbm, out_hbm, idx_v, tbl_v, acc_v):
        wid = lax.axis_index("s") * nc + lax.axis_index("c")
        pltpu.sync_copy(tbl_hbm, tbl_v)
        pltpu.sync_copy(idx_hbm.at[pl.ds(wid * per_w, per_w)], idx_v)

        unroll = 2

        def body(j, accs):
            base = j * (unroll * 2 * lanes)
            out = []
            for u in range(unroll):
                packed = plsc.bitcast(
                    idx_v[pl.ds(base + u * 2 * lanes, 2 * lanes)], jnp.int32)
                lo = packed & 0xFFFF
                hi = lax.shift_right_logical(packed, 16)
                out.append(accs[2 * u] + plsc.load_gather(tbl_v, [lo]))
                out.append(accs[2 * u + 1] + plsc.load_gather(tbl_v, [hi]))
            return tuple(out)

        z = jnp.zeros((lanes,), jnp.float32)
        accs = lax.fori_loop(0, per_w // (unroll * 2 * lanes), body,
                             (z,) * (2 * unroll))
        acc_v[...] = (accs[0] + accs[1]) + (accs[2] + accs[3])
        pltpu.sync_copy(acc_v, out_hbm.at[pl.ds(wid * lanes, lanes)])

    return sc_reduce


def kernel(input, y_true, D):
    b, c, h, w = input.shape
    n = b * h * w
    idx = _cost_index(input, y_true).reshape(n)
    tbl = jnp.zeros((_TBL,), jnp.float32).at[: c * c].set(D.reshape(-1))
    partials = _make_sc_reduce(n)(idx, tbl)
    return jnp.sum(partials) / n
